# baseline (speedup 1.0000x reference)
import functools

import jax
import jax.numpy as jnp
from jax.experimental import pallas as pl
from jax.experimental.pallas import tpu as pltpu
from jax.experimental.pallas import tpu_sc as plsc

_SCALES = (1, 5, 15, 60)

_GATHER_WINDOW = 128
_TC_BLOCK_B = 32
_NUM_CHUNKS = 4


def _build_combined(max_seq, embs, w):
    parts = []
    for i, (emb, scale) in enumerate(zip(embs, _SCALES)):
        rep = jnp.repeat(emb, scale, axis=0)[:max_seq]
        parts.append(rep * w[i])
    return jnp.concatenate(parts, axis=1)


def _sc_gather(table, idx_flat, n_rows, d):
    mesh = plsc.VectorSubcoreMesh(core_axis_name="c", subcore_axis_name="s")

    @functools.partial(
        pl.kernel,
        out_type=jax.ShapeDtypeStruct((n_rows, d), table.dtype),
        mesh=mesh,
    )
    def gather_kernel(tbl_hbm, idx_hbm, out_hbm):
        def body(idx_vmem, out_vmem):
            pltpu.sync_copy(tbl_hbm.at[idx_vmem.at[0]], out_vmem)

        pltpu.emit_pipeline(
            body,
            grid=(n_rows // _GATHER_WINDOW,),
            in_specs=[pl.BlockSpec((1, _GATHER_WINDOW), lambda i: (0, i))],
            out_specs=[pl.BlockSpec((_GATHER_WINDOW, d), lambda i: (i, 0))],
            core_axis_name=("c", "s"),
            dimension_semantics=(pltpu.PARALLEL,),
        )(idx_hbm, out_hbm)

    return gather_kernel(table, idx_flat)


def _tc_add_chunk(buf, x, g_chunk, pe_s, chunk, blocks_per_chunk):
    b, s, d = x.shape
    base = chunk * blocks_per_chunk

    def body(buf_ref, x_ref, g_ref, pe_ref, o_ref):
        del buf_ref
        o_ref[...] = x_ref[...] + (pe_ref[...] + g_ref[...])

    first = buf is None
    in_specs = [
        pl.BlockSpec(memory_space=pl.ANY),
        pl.BlockSpec((_TC_BLOCK_B, s, d), lambda i: (base + i, 0, 0)),
        pl.BlockSpec((_TC_BLOCK_B, s, d), lambda i: (i, 0, 0)),
        pl.BlockSpec((s, d), lambda i: (0, 0)),
    ]
    operands = [x if first else buf, x, g_chunk, pe_s]
    return pl.pallas_call(
        body,
        out_shape=jax.ShapeDtypeStruct((b, s, d), x.dtype),
        grid=(blocks_per_chunk,),
        in_specs=in_specs,
        out_specs=pl.BlockSpec((_TC_BLOCK_B, s, d), lambda i: (base + i, 0, 0)),
        input_output_aliases={} if first else {0: 0},
    )(*operands)


def kernel(x, time_indices, pe, emb_1, emb_5, emb_15, emb_60, temporal_importance):
    b, s, d = x.shape
    max_seq = pe.shape[0]
    combined = _build_combined(
        max_seq, (emb_1, emb_5, emb_15, emb_60), temporal_importance
    )
    idx = time_indices.reshape(1, b * s).astype(jnp.int32)
    pe_s = pe[:s]

    chunk_b = b // _NUM_CHUNKS
    chunk_n = chunk_b * s
    blocks_per_chunk = chunk_b // _TC_BLOCK_B

    gathered = [
        _sc_gather(
            combined,
            jax.lax.dynamic_slice(idx, (0, k * chunk_n), (1, chunk_n)),
            chunk_n,
            d,
        ).reshape(chunk_b, s, d)
        for k in range(_NUM_CHUNKS)
    ]
    buf = None
    for k in range(_NUM_CHUNKS):
        buf = _tc_add_chunk(buf, x, gathered[k], pe_s, k, blocks_per_chunk)
    return buf


# speedup vs baseline: 1.6967x; 1.6967x over previous
import functools

import jax
from jax import lax
import jax.numpy as jnp
from jax.experimental import pallas as pl
from jax.experimental.pallas import tpu as pltpu
from jax.experimental.pallas import tpu_sc as plsc

_SCALES = (1, 5, 15, 60)

_GATHER_WINDOW = 128
_TC_BLOCK_B = 32


def _build_combined(max_seq, embs, w):
    parts = []
    for i, (emb, scale) in enumerate(zip(embs, _SCALES)):
        rep = jnp.repeat(emb, scale, axis=0)[:max_seq]
        parts.append(rep * w[i])
    return jnp.concatenate(parts, axis=1)


def _sc_gather(table, idx_flat, n_rows, d):
    mesh = plsc.VectorSubcoreMesh(core_axis_name="c", subcore_axis_name="s")
    n_tbl = table.shape[0]

    @functools.partial(
        pl.kernel,
        out_type=jax.ShapeDtypeStruct((n_rows, d), table.dtype),
        mesh=mesh,
        scratch_types=[pltpu.VMEM_SHARED((n_tbl, d), table.dtype)],
    )
    def gather_kernel(tbl_hbm, idx_hbm, out_hbm, tbl_sh):
        @pl.when(lax.axis_index("s") == 0)
        def _():
            pltpu.sync_copy(tbl_hbm, tbl_sh)

        plsc.subcore_barrier()

        def body(idx_vmem, out_vmem):
            pltpu.sync_copy(tbl_sh.at[idx_vmem.at[0]], out_vmem)

        pltpu.emit_pipeline(
            body,
            grid=(n_rows // _GATHER_WINDOW,),
            in_specs=[pl.BlockSpec((1, _GATHER_WINDOW), lambda i: (0, i))],
            out_specs=[pl.BlockSpec((_GATHER_WINDOW, d), lambda i: (i, 0))],
            core_axis_name=("c", "s"),
            dimension_semantics=(pltpu.PARALLEL,),
        )(idx_hbm, out_hbm)

    return gather_kernel(table, idx_flat)


def _tc_add(x, g, pe_s):
    b, s, d = x.shape

    def body(x_ref, g_ref, pe_ref, o_ref):
        o_ref[...] = x_ref[...] + (pe_ref[...] + g_ref[...])

    return pl.pallas_call(
        body,
        out_shape=jax.ShapeDtypeStruct((b, s, d), x.dtype),
        grid=(b // _TC_BLOCK_B,),
        in_specs=[
            pl.BlockSpec((_TC_BLOCK_B, s, d), lambda i: (i, 0, 0)),
            pl.BlockSpec((_TC_BLOCK_B, s, d), lambda i: (i, 0, 0)),
            pl.BlockSpec((s, d), lambda i: (0, 0)),
        ],
        out_specs=pl.BlockSpec((_TC_BLOCK_B, s, d), lambda i: (i, 0, 0)),
    )(x, g, pe_s)


def kernel(x, time_indices, pe, emb_1, emb_5, emb_15, emb_60, temporal_importance):
    b, s, d = x.shape
    max_seq = pe.shape[0]
    combined = _build_combined(
        max_seq, (emb_1, emb_5, emb_15, emb_60), temporal_importance
    )
    idx = time_indices.reshape(1, b * s).astype(jnp.int32)
    g = _sc_gather(combined, idx, b * s, d).reshape(b, s, d)
    return _tc_add(x, g, pe[:s])


# speedup vs baseline: 1.7210x; 1.0143x over previous
import functools

import jax
from jax import lax
import jax.numpy as jnp
from jax.experimental import pallas as pl
from jax.experimental.pallas import tpu as pltpu
from jax.experimental.pallas import tpu_sc as plsc

_SCALES = (1, 5, 15, 60)

_GATHER_WINDOW = 256
_TC_BLOCK_B = 64


def _build_combined(max_seq, embs, w):
    parts = []
    for i, (emb, scale) in enumerate(zip(embs, _SCALES)):
        rep = jnp.repeat(emb, scale, axis=0)[:max_seq]
        parts.append(rep * w[i])
    return jnp.concatenate(parts, axis=1)


def _sc_gather(table, idx_flat, n_rows, d):
    mesh = plsc.VectorSubcoreMesh(core_axis_name="c", subcore_axis_name="s")
    n_tbl = table.shape[0]

    @functools.partial(
        pl.kernel,
        out_type=jax.ShapeDtypeStruct((n_rows, d), table.dtype),
        mesh=mesh,
        scratch_types=[pltpu.VMEM_SHARED((n_tbl, d), table.dtype)],
    )
    def gather_kernel(tbl_hbm, idx_hbm, out_hbm, tbl_sh):
        @pl.when(lax.axis_index("s") == 0)
        def _():
            pltpu.sync_copy(tbl_hbm, tbl_sh)

        plsc.subcore_barrier()

        def body(idx_vmem, out_vmem):
            pltpu.sync_copy(tbl_sh.at[idx_vmem.at[0]], out_vmem)

        pltpu.emit_pipeline(
            body,
            grid=(n_rows // _GATHER_WINDOW,),
            in_specs=[pl.BlockSpec((1, _GATHER_WINDOW), lambda i: (0, i))],
            out_specs=[pl.BlockSpec((_GATHER_WINDOW, d), lambda i: (i, 0))],
            core_axis_name=("c", "s"),
            dimension_semantics=(pltpu.PARALLEL,),
        )(idx_hbm, out_hbm)

    return gather_kernel(table, idx_flat)


def _tc_add(x, g, pe_s):
    b, s, d = x.shape

    def body(x_ref, g_ref, pe_ref, o_ref):
        o_ref[...] = x_ref[...] + (pe_ref[...] + g_ref[...])

    return pl.pallas_call(
        body,
        out_shape=jax.ShapeDtypeStruct((b, s, d), x.dtype),
        grid=(b // _TC_BLOCK_B,),
        in_specs=[
            pl.BlockSpec((_TC_BLOCK_B, s, d), lambda i: (i, 0, 0)),
            pl.BlockSpec((_TC_BLOCK_B, s, d), lambda i: (i, 0, 0)),
            pl.BlockSpec((s, d), lambda i: (0, 0)),
        ],
        out_specs=pl.BlockSpec((_TC_BLOCK_B, s, d), lambda i: (i, 0, 0)),
    )(x, g, pe_s)


def kernel(x, time_indices, pe, emb_1, emb_5, emb_15, emb_60, temporal_importance):
    b, s, d = x.shape
    max_seq = pe.shape[0]
    combined = _build_combined(
        max_seq, (emb_1, emb_5, emb_15, emb_60), temporal_importance
    )
    idx = time_indices.reshape(1, b * s).astype(jnp.int32)
    g = _sc_gather(combined, idx, b * s, d).reshape(b, s, d)
    return _tc_add(x, g, pe[:s])


# speedup vs baseline: 1.7340x; 1.0075x over previous
import functools

import jax
from jax import lax
import jax.numpy as jnp
from jax.experimental import pallas as pl
from jax.experimental.pallas import tpu as pltpu
from jax.experimental.pallas import tpu_sc as plsc

_SCALES = (1, 5, 15, 60)

_GATHER_WINDOW = 256
_TC_BLOCK_B = 64


def _build_combined(max_seq, embs, w):
    parts = []
    for i, (emb, scale) in enumerate(zip(embs, _SCALES)):
        rep = jnp.repeat(emb, scale, axis=0)[:max_seq]
        parts.append(rep * w[i])
    return jnp.concatenate(parts, axis=1)


def _sc_gather(table, idx_flat, n_rows, d):
    mesh = plsc.VectorSubcoreMesh(core_axis_name="c", subcore_axis_name="s")
    n_tbl = table.shape[0]

    @functools.partial(
        pl.kernel,
        out_type=jax.ShapeDtypeStruct((n_rows, d), table.dtype),
        mesh=mesh,
        scratch_types=[pltpu.VMEM_SHARED((n_tbl, d), table.dtype)],
    )
    def gather_kernel(tbl_hbm, idx_hbm, out_hbm, tbl_sh):
        @pl.when(lax.axis_index("s") == 0)
        def _():
            pltpu.sync_copy(tbl_hbm, tbl_sh)

        plsc.subcore_barrier()

        def body(idx_vmem, out_vmem):
            pltpu.sync_copy(tbl_sh.at[idx_vmem.at[0]], out_vmem)

        pltpu.emit_pipeline(
            body,
            grid=(n_rows // _GATHER_WINDOW,),
            in_specs=[pl.BlockSpec((1, _GATHER_WINDOW), lambda i: (0, i))],
            out_specs=[pl.BlockSpec((_GATHER_WINDOW, d), lambda i: (i, 0))],
            core_axis_name=("c", "s"),
            dimension_semantics=(pltpu.PARALLEL,),
        )(idx_hbm, out_hbm)

    return gather_kernel(table, idx_flat)


def _tc_add_chunk(buf, x, g_chunk, pe_s, chunk, blocks_per_chunk):
    b, s, d = x.shape
    base = chunk * blocks_per_chunk

    def body(buf_ref, x_ref, g_ref, pe_ref, o_ref):
        del buf_ref
        o_ref[...] = x_ref[...] + (pe_ref[...] + g_ref[...])

    first = buf is None
    in_specs = [
        pl.BlockSpec(memory_space=pl.ANY),
        pl.BlockSpec((_TC_BLOCK_B, s, d), lambda i: (base + i, 0, 0)),
        pl.BlockSpec((_TC_BLOCK_B, s, d), lambda i: (i, 0, 0)),
        pl.BlockSpec((s, d), lambda i: (0, 0)),
    ]
    return pl.pallas_call(
        body,
        out_shape=jax.ShapeDtypeStruct((b, s, d), x.dtype),
        grid=(blocks_per_chunk,),
        in_specs=in_specs,
        out_specs=pl.BlockSpec((_TC_BLOCK_B, s, d), lambda i: (base + i, 0, 0)),
        input_output_aliases={} if first else {0: 0},
    )(x if first else buf, x, g_chunk, pe_s)


_NUM_CHUNKS = 4


def kernel(x, time_indices, pe, emb_1, emb_5, emb_15, emb_60, temporal_importance):
    b, s, d = x.shape
    max_seq = pe.shape[0]
    combined = _build_combined(
        max_seq, (emb_1, emb_5, emb_15, emb_60), temporal_importance
    )
    idx = time_indices.reshape(1, b * s).astype(jnp.int32)
    pe_s = pe[:s]

    chunk_b = b // _NUM_CHUNKS
    chunk_n = chunk_b * s
    blocks_per_chunk = chunk_b // _TC_BLOCK_B

    gathered = [
        _sc_gather(
            combined,
            lax.dynamic_slice(idx, (0, k * chunk_n), (1, chunk_n)),
            chunk_n,
            d,
        ).reshape(chunk_b, s, d)
        for k in range(_NUM_CHUNKS)
    ]
    buf = None
    for k in range(_NUM_CHUNKS):
        buf = _tc_add_chunk(buf, x, gathered[k], pe_s, k, blocks_per_chunk)
    return buf


# speedup vs baseline: 2.2023x; 1.2701x over previous
import functools

import jax
from jax import lax
import jax.numpy as jnp
from jax.experimental import pallas as pl
from jax.experimental.pallas import tpu as pltpu
from jax.experimental.pallas import tpu_sc as plsc

_SCALES = (1, 5, 15, 60)

_SC_CORES = 2
_SC_SUBCORES = 16
_NBUF = 4


def _build_table(max_seq, embs, w, pe_s):
    parts = []
    for i, (emb, scale) in enumerate(zip(embs, _SCALES)):
        rep = jnp.repeat(emb, scale, axis=0)[:max_seq]
        parts.append(rep * w[i])
    return jnp.concatenate([jnp.concatenate(parts, axis=1), pe_s], axis=0)


def _sc_fused(table, x2d, idx_flat, pidx_flat, n_rows, d, chunk):
    mesh = plsc.VectorSubcoreMesh(core_axis_name="c", subcore_axis_name="s")
    n_tbl = table.shape[0]
    n_workers = _SC_CORES * _SC_SUBCORES
    per_w = n_rows // n_workers
    n_chunks = per_w // chunk
    assert per_w % chunk == 0 and n_chunks % _NBUF == 0

    @functools.partial(
        pl.kernel,
        out_type=jax.ShapeDtypeStruct((n_rows, d), x2d.dtype),
        mesh=mesh,
        scratch_types=[
            pltpu.VMEM_SHARED((n_tbl, d), table.dtype),
            pltpu.VMEM((_NBUF, chunk, d), x2d.dtype),
            pltpu.VMEM((_NBUF, chunk), jnp.int32),
            pltpu.VMEM((_NBUF, chunk), jnp.int32),
        ]
        + [pltpu.SemaphoreType.DMA] * (4 * _NBUF),
    )
    def fused_kernel(tbl_hbm, x_hbm, idx_hbm, pidx_hbm, out_hbm,
                     tbl_sh, xbuf, ibuf, pbuf, *sems):
        xsems = sems[:_NBUF]
        isems = sems[_NBUF:2 * _NBUF]
        psems = sems[2 * _NBUF:3 * _NBUF]
        osems = sems[3 * _NBUF:]

        @pl.when(lax.axis_index("s") == 0)
        def _():
            pltpu.sync_copy(tbl_hbm, tbl_sh)

        plsc.subcore_barrier()

        wid = lax.axis_index("s") * _SC_CORES + lax.axis_index("c")
        base = wid * per_w

        def start_in(j, b):
            pltpu.async_copy(x_hbm.at[pl.ds(base + j * chunk, chunk)],
                             xbuf.at[b], xsems[b])
            pltpu.async_copy(idx_hbm.at[0, pl.ds(base + j * chunk, chunk)],
                             ibuf.at[b], isems[b])
            pltpu.async_copy(pidx_hbm.at[0, pl.ds(base + j * chunk, chunk)],
                             pbuf.at[b], psems[b])

        def wait_in(b):
            pltpu.make_async_copy(x_hbm.at[pl.ds(base, chunk)],
                                  xbuf.at[b], xsems[b]).wait()
            pltpu.make_async_copy(idx_hbm.at[0, pl.ds(0, chunk)],
                                  ibuf.at[b], isems[b]).wait()
            pltpu.make_async_copy(pidx_hbm.at[0, pl.ds(0, chunk)],
                                  pbuf.at[b], psems[b]).wait()

        def drain_out(b):
            pltpu.make_async_copy(xbuf.at[b],
                                  out_hbm.at[pl.ds(base, chunk)],
                                  osems[b]).wait()

        start_in(0, 0)
        start_in(1, 1)

        @pl.loop(0, n_chunks, step=_NBUF)
        def _(j0):
            for b in range(_NBUF):
                j = j0 + b
                wait_in(b)
                pltpu.sync_copy(tbl_sh.at[ibuf.at[b]], xbuf.at[b], add=True)
                pltpu.sync_copy(tbl_sh.at[pbuf.at[b]], xbuf.at[b], add=True)
                pltpu.async_copy(xbuf.at[b],
                                 out_hbm.at[pl.ds(base + j * chunk, chunk)],
                                 osems[b])
                b2 = (b + 2) % _NBUF

                @pl.when(j >= 2)
                def _():
                    drain_out(b2)

                @pl.when(j + 2 < n_chunks)
                def _():
                    start_in(j + 2, b2)

        drain_out((n_chunks - 2) % _NBUF)
        drain_out((n_chunks - 1) % _NBUF)

    return fused_kernel(table, x2d, idx_flat, pidx_flat)


def kernel(x, time_indices, pe, emb_1, emb_5, emb_15, emb_60, temporal_importance):
    b, s, d = x.shape
    max_seq = pe.shape[0]
    table = _build_table(
        max_seq, (emb_1, emb_5, emb_15, emb_60), temporal_importance, pe[:s]
    )
    idx = time_indices.reshape(1, b * s).astype(jnp.int32)
    pe_idx = jnp.broadcast_to(
        max_seq + jnp.arange(s, dtype=jnp.int32), (b, s)
    ).reshape(1, b * s)
    out = _sc_fused(table, x.reshape(b * s, d), idx, pe_idx, b * s, d, 128)
    return out.reshape(b, s, d)
